# baseline (speedup 1.0000x reference)
import jax
import jax.numpy as jnp
from jax.experimental import pallas as pl

NNODES = 10000
NREL = 8
NFEAT = 128
NHID = 64
NCLASS = 16
R = 2 * NREL + 1


def _log_softmax_body(x_ref, o_ref):
    x = x_ref[...]
    m = jnp.max(x, axis=1, keepdims=True)
    e = jnp.exp(x - m)
    s = jnp.sum(e, axis=1, keepdims=True)
    o_ref[...] = x - m - jnp.log(s)


def _rgcn_layer(X, W, b, tp):
    s, r, o = tp[:, 0], tp[:, 1], tp[:, 2]
    XW = jnp.einsum('nf,rfh->rnh', X, W)
    msgs = XW[r, s]
    key = o * R + r
    counts = jax.ops.segment_sum(jnp.ones(key.shape, jnp.float32), key, num_segments=NNODES * R)
    norm = 1.0 / jnp.maximum(counts[key], 1.0)
    out = jax.ops.segment_sum(msgs * norm[:, None], o, num_segments=NNODES)
    return out + b


def kernel(triples, X0, W1, b1, W2, b2):
    s, r, o = triples[:, 0], triples[:, 1], triples[:, 2]
    inv = jnp.stack([o, r + NREL, s], axis=1)
    nodes = jnp.arange(NNODES, dtype=triples.dtype)
    self_loops = jnp.stack([nodes, jnp.full((NNODES,), 2 * NREL, dtype=triples.dtype), nodes], axis=1)
    tp = jnp.concatenate([triples, inv, self_loops], axis=0)
    h = jax.nn.relu(_rgcn_layer(X0, W1, b1, tp))
    logits = _rgcn_layer(h, W2, b2, tp)
    return pl.pallas_call(
        _log_softmax_body,
        out_shape=jax.ShapeDtypeStruct((NNODES, NCLASS), jnp.float32),
    )(logits)


# speedup vs baseline: 39.5896x; 39.5896x over previous
import functools

import jax
import jax.numpy as jnp
from jax import lax
from jax.experimental import pallas as pl
from jax.experimental.pallas import tpu as pltpu
from jax.experimental.pallas import tpu_sc as plsc

N = 10000
NREL = 8
NFEAT = 128
NHID = 64
NCLASS = 16
R = 2 * NREL + 1
E = 320000

NC = 2
NS = 16
L = 16
NW = NC * NS

NPAD = 10240
EPAD = 327680
CSH = NPAD * R
STRIPE = CSH // NS
ROWS_T = NPAD // NS

B = 128
KC = 256
NB = KC // B
T1 = EPAD // NS
T3 = EPAD // NW


def _sc_layer_body(compute_counts, H, *refs):
    if compute_counts:
        (s_hbm, r_hbm, o_hbm, xw_hbm,
         acc_out, norm_out,
         s_b, r_b, o_b, kbuf, ones, rid2, key2, dst2, normv, rows,
         stripe, csh, acc, sem, sem2) = refs
    else:
        (s_hbm, r_hbm, o_hbm, xw_hbm, norm_in,
         acc_out,
         s_b, r_b, o_b, kbuf, ones, rid2, key2, dst2, normv, rows,
         stripe, csh, acc, sem, sem2) = refs

    cid = lax.axis_index("c")
    sid = lax.axis_index("s")

    def zrow(i, c):
        for k in range(H // L):
            rows[0, i, pl.ds(k * L, L)] = jnp.zeros((L,), jnp.float32)
        return c
    lax.fori_loop(0, B, zrow, 0)

    def zstripe(i, c):
        stripe[pl.ds(i * L, L)] = jnp.zeros((L,), jnp.float32)
        return c
    lax.fori_loop(0, STRIPE // L, zstripe, 0)

    def ofill(i, c):
        ones[pl.ds(i * L, L)] = jnp.ones((L,), jnp.float32)
        return c
    lax.fori_loop(0, B // L, ofill, 0)

    for q in range(ROWS_T // B):
        pltpu.sync_copy(rows.at[0], acc.at[pl.ds(sid * ROWS_T + q * B, B), :])

    if compute_counts:
        pltpu.sync_copy(stripe, csh.at[pl.ds(sid * STRIPE, STRIPE)])
    else:
        pltpu.sync_copy(norm_in.at[pl.ds(sid * STRIPE, STRIPE)], stripe)
        pltpu.sync_copy(stripe, csh.at[pl.ds(sid * STRIPE, STRIPE)])
    plsc.subcore_barrier()

    if compute_counts:
        def p1(c, carry):
            base = sid * T1 + c * KC
            pltpu.sync_copy(s_hbm.at[pl.ds(base, KC)], s_b)
            pltpu.sync_copy(r_hbm.at[pl.ds(base, KC)], r_b)
            pltpu.sync_copy(o_hbm.at[pl.ds(base, KC)], o_b)

            def kcomp(j, cc):
                blk = j // (B // L)
                off = (j % (B // L)) * L
                sv = s_b[pl.ds(j * L, L)]
                rv = r_b[pl.ds(j * L, L)]
                ov = o_b[pl.ds(j * L, L)]
                kbuf[blk, pl.ds(off, L)] = ov * R + rv
                kbuf[NB + blk, pl.ds(off, L)] = sv * R + rv + NREL
                return cc
            lax.fori_loop(0, KC // L, kcomp, 0)
            for j in range(2 * NB):
                pltpu.sync_copy(ones, csh.at[kbuf.at[j]], add=True)
            return carry
        lax.fori_loop(0, T1 // KC, p1, 0)
        plsc.subcore_barrier()

        pltpu.sync_copy(csh.at[pl.ds(sid * STRIPE, STRIPE)], stripe)

        def nconv(i, c):
            cv = stripe[pl.ds(i * L, L)]
            stripe[pl.ds(i * L, L)] = 1.0 / jnp.maximum(cv, 1.0)
            return c
        lax.fori_loop(0, STRIPE // L, nconv, 0)
        pltpu.sync_copy(stripe, csh.at[pl.ds(sid * STRIPE, STRIPE)])

        @pl.when(cid == 0)
        def _():
            pltpu.sync_copy(stripe, norm_out.at[pl.ds(sid * STRIPE, STRIPE)])
        plsc.subcore_barrier()

    wid = cid * NS + sid

    def p3(c, carry):
        base = wid * T3 + c * KC
        pltpu.sync_copy(s_hbm.at[pl.ds(base, KC)], s_b)
        pltpu.sync_copy(r_hbm.at[pl.ds(base, KC)], r_b)
        pltpu.sync_copy(o_hbm.at[pl.ds(base, KC)], o_b)

        def icomp(j, cc):
            blk = j // (B // L)
            off = (j % (B // L)) * L
            sv = s_b[pl.ds(j * L, L)]
            rv = r_b[pl.ds(j * L, L)]
            ov = o_b[pl.ds(j * L, L)]
            a = sv * R + rv
            b = ov * R + rv
            rid2[blk, pl.ds(off, L)] = a
            rid2[NB + blk, pl.ds(off, L)] = b + NREL
            key2[blk, pl.ds(off, L)] = b
            key2[NB + blk, pl.ds(off, L)] = a + NREL
            dst2[blk, pl.ds(off, L)] = ov
            dst2[NB + blk, pl.ds(off, L)] = sv
            return cc
        lax.fori_loop(0, KC // L, icomp, 0)

        handles = []
        for j in range(2 * NB):
            handles.append(pltpu.async_copy(
                xw_hbm.at[rid2.at[j]], rows.at[j], sem))
            handles.append(pltpu.async_copy(
                csh.at[key2.at[j]], normv.at[j], sem2))
        for h in handles:
            h.wait()

        def scale(g, cc):
            blk = g // (B // L)
            off = (g % (B // L)) * L
            nv = normv[blk, pl.ds(off, L)]
            for i in range(L):
                nrm = nv[i]
                for k in range(H // L):
                    rows[blk, off + i, pl.ds(k * L, L)] = (
                        rows[blk, off + i, pl.ds(k * L, L)] * nrm)
            return cc
        lax.fori_loop(0, (2 * KC) // L, scale, 0)

        for j in range(2 * NB):
            pltpu.sync_copy(rows.at[j], acc.at[dst2.at[j]], add=True)
        return carry
    lax.fori_loop(0, T3 // KC, p3, 0)
    plsc.subcore_barrier()

    pltpu.sync_copy(acc.at[pl.ds(sid * ROWS_T, ROWS_T), :],
                    acc_out.at[cid, pl.ds(sid * ROWS_T, ROWS_T), :])


def _make_sc_layer(H, compute_counts):
    mesh = plsc.VectorSubcoreMesh(core_axis_name="c", subcore_axis_name="s")
    if compute_counts:
        out_type = [jax.ShapeDtypeStruct((NC, NPAD, H), jnp.float32),
                    jax.ShapeDtypeStruct((CSH,), jnp.float32)]
    else:
        out_type = jax.ShapeDtypeStruct((NC, NPAD, H), jnp.float32)
    scratch = [
        pltpu.VMEM((KC,), jnp.int32),
        pltpu.VMEM((KC,), jnp.int32),
        pltpu.VMEM((KC,), jnp.int32),
        pltpu.VMEM((2 * NB, B), jnp.int32),
        pltpu.VMEM((B,), jnp.float32),
        pltpu.VMEM((2 * NB, B), jnp.int32),
        pltpu.VMEM((2 * NB, B), jnp.int32),
        pltpu.VMEM((2 * NB, B), jnp.int32),
        pltpu.VMEM((2 * NB, B), jnp.float32),
        pltpu.VMEM((2 * NB, B, H), jnp.float32),
        pltpu.VMEM((STRIPE,), jnp.float32),
        pltpu.VMEM_SHARED((CSH,), jnp.float32),
        pltpu.VMEM_SHARED((NPAD, H), jnp.float32),
        pltpu.SemaphoreType.DMA,
        pltpu.SemaphoreType.DMA,
    ]
    body = functools.partial(_sc_layer_body, compute_counts, H)
    return pl.kernel(
        body, out_type=out_type, mesh=mesh, scratch_types=scratch,
        compiler_params=pltpu.CompilerParams(use_tc_tiling_on_sc=False))


def _mm_body(x_ref, w_ref, o_ref):
    o_ref[...] = jnp.dot(x_ref[...], w_ref[...], preferred_element_type=jnp.float32)


def _mm(X, Wcat):
    n, f = X.shape
    hh = Wcat.shape[1]
    bn = 1024
    return pl.pallas_call(
        _mm_body,
        grid=(n // bn,),
        in_specs=[pl.BlockSpec((bn, f), lambda i: (i, 0)),
                  pl.BlockSpec((f, hh), lambda i: (0, 0))],
        out_specs=pl.BlockSpec((bn, hh), lambda i: (i, 0)),
        out_shape=jax.ShapeDtypeStruct((n, hh), jnp.float32),
    )(X, Wcat)


def _cmb_mm_body(p_ref, self_ref, b_ref, w_ref, o_ref):
    h = p_ref[0] + p_ref[1] + self_ref[...] + b_ref[...]
    h = jnp.maximum(h, 0.0)
    o_ref[...] = jnp.dot(h, w_ref[...], preferred_element_type=jnp.float32)


def _cmb_mm(parts, selfrows, bvec, Wcat):
    n, f = selfrows.shape
    hh = Wcat.shape[1]
    bn = 1024
    return pl.pallas_call(
        _cmb_mm_body,
        grid=(n // bn,),
        in_specs=[pl.BlockSpec((NC, bn, f), lambda i: (0, i, 0)),
                  pl.BlockSpec((bn, f), lambda i: (i, 0)),
                  pl.BlockSpec((1, f), lambda i: (0, 0)),
                  pl.BlockSpec((f, hh), lambda i: (0, 0))],
        out_specs=pl.BlockSpec((bn, hh), lambda i: (i, 0)),
        out_shape=jax.ShapeDtypeStruct((n, hh), jnp.float32),
    )(parts, selfrows, bvec, Wcat)


def _final_body(p_ref, self_ref, b_ref, o_ref):
    x = p_ref[0] + p_ref[1] + self_ref[...] + b_ref[...]
    m = jnp.max(x, axis=1, keepdims=True)
    e = jnp.exp(x - m)
    ssum = jnp.sum(e, axis=1, keepdims=True)
    o_ref[...] = x - m - jnp.log(ssum)


def _final(parts, selfrows, bvec):
    n, f = selfrows.shape
    bn = 1024
    return pl.pallas_call(
        _final_body,
        grid=(n // bn,),
        in_specs=[pl.BlockSpec((NC, bn, f), lambda i: (0, i, 0)),
                  pl.BlockSpec((bn, f), lambda i: (i, 0)),
                  pl.BlockSpec((1, f), lambda i: (0, 0))],
        out_specs=pl.BlockSpec((bn, f), lambda i: (i, 0)),
        out_shape=jax.ShapeDtypeStruct((n, f), jnp.float32),
    )(parts, selfrows, bvec)


_sc_layer1 = _make_sc_layer(NHID, True)
_sc_layer2 = _make_sc_layer(NCLASS, False)


def kernel(triples, X0, W1, b1, W2, b2):
    s = triples[:, 0]
    r = triples[:, 1]
    o = triples[:, 2]
    npade = EPAD - E
    padn = (N + (jnp.arange(npade, dtype=jnp.int32) % (NPAD - N))).astype(s.dtype)
    sp = jnp.concatenate([s, padn])
    rp = jnp.concatenate([r, jnp.zeros((npade,), s.dtype)])
    op = jnp.concatenate([o, padn])

    X0p = jnp.pad(X0, ((0, NPAD - N), (0, 0)))
    W1cat = W1.transpose(1, 0, 2).reshape(NFEAT, R * NHID)
    W2cat = W2.transpose(1, 0, 2).reshape(NHID, R * NCLASS)

    xw1 = _mm(X0p, W1cat)
    xw1_flat = xw1.reshape(NPAD * R, NHID)
    self1 = xw1.reshape(NPAD, R, NHID)[:, 2 * NREL]

    p1, norm = _sc_layer1(sp, rp, op, xw1_flat)

    hxw2 = _cmb_mm(p1, self1, b1.reshape(1, -1), W2cat)
    xw2_flat = hxw2.reshape(NPAD * R, NCLASS)
    self2 = hxw2.reshape(NPAD, R, NCLASS)[:, 2 * NREL]

    p2 = _sc_layer2(sp, rp, op, xw2_flat, norm)

    out = _final(p2, self2, b2.reshape(1, -1))
    return out[:N]


# speedup vs baseline: 40.0582x; 1.0118x over previous
import functools

import jax
import jax.numpy as jnp
from jax import lax
from jax.experimental import pallas as pl
from jax.experimental.pallas import tpu as pltpu
from jax.experimental.pallas import tpu_sc as plsc

N = 10000
NREL = 8
NFEAT = 128
NHID = 64
NCLASS = 16
R = 2 * NREL + 1
E = 320000

NC = 2
NS = 16
L = 16
NW = NC * NS

NPAD = 10240
EPAD = 327680
CSH = NPAD * R
STRIPE = CSH // NS
ROWS_T = NPAD // NS

B = 128
KC = 256
NB = KC // B
T1 = EPAD // NS
T3 = EPAD // NW


def _sc_layer_body(compute_counts, H, *refs):
    if compute_counts:
        (s_hbm, r_hbm, o_hbm, xw_hbm,
         acc_out, norm_out,
         s_b, r_b, o_b, kbuf, ones, rid2, key2, dst2, normv, rows,
         stripe, csh, acc, sem, sem2, sem3, sem4) = refs
    else:
        (s_hbm, r_hbm, o_hbm, xw_hbm, norm_in,
         acc_out,
         s_b, r_b, o_b, kbuf, ones, rid2, key2, dst2, normv, rows,
         stripe, csh, acc, sem, sem2, sem3, sem4) = refs

    cid = lax.axis_index("c")
    sid = lax.axis_index("s")

    def zrow(i, c):
        for k in range(H // L):
            rows[0, i, pl.ds(k * L, L)] = jnp.zeros((L,), jnp.float32)
        return c
    lax.fori_loop(0, B, zrow, 0)

    def zstripe(i, c):
        stripe[pl.ds(i * L, L)] = jnp.zeros((L,), jnp.float32)
        return c
    lax.fori_loop(0, STRIPE // L, zstripe, 0)

    def ofill(i, c):
        ones[pl.ds(i * L, L)] = jnp.ones((L,), jnp.float32)
        return c
    lax.fori_loop(0, B // L, ofill, 0)

    for q in range(ROWS_T // B):
        pltpu.sync_copy(rows.at[0], acc.at[pl.ds(sid * ROWS_T + q * B, B), :])

    if compute_counts:
        pltpu.sync_copy(stripe, csh.at[pl.ds(sid * STRIPE, STRIPE)])
    else:
        pltpu.sync_copy(norm_in.at[pl.ds(sid * STRIPE, STRIPE)], stripe)
        pltpu.sync_copy(stripe, csh.at[pl.ds(sid * STRIPE, STRIPE)])
    plsc.subcore_barrier()

    if compute_counts:
        def p1(c, carry):
            base = sid * T1 + c * KC
            pltpu.sync_copy(s_hbm.at[pl.ds(base, KC)], s_b)
            pltpu.sync_copy(r_hbm.at[pl.ds(base, KC)], r_b)
            pltpu.sync_copy(o_hbm.at[pl.ds(base, KC)], o_b)

            def kcomp(j, cc):
                blk = j // (B // L)
                off = (j % (B // L)) * L
                sv = s_b[pl.ds(j * L, L)]
                rv = r_b[pl.ds(j * L, L)]
                ov = o_b[pl.ds(j * L, L)]
                kbuf[blk, pl.ds(off, L)] = ov * R + rv
                kbuf[NB + blk, pl.ds(off, L)] = sv * R + rv + NREL
                return cc
            lax.fori_loop(0, KC // L, kcomp, 0)
            for j in range(2 * NB):
                pltpu.sync_copy(ones, csh.at[kbuf.at[j]], add=True)
            return carry
        lax.fori_loop(0, T1 // KC, p1, 0)
        plsc.subcore_barrier()

        pltpu.sync_copy(csh.at[pl.ds(sid * STRIPE, STRIPE)], stripe)

        def nconv(i, c):
            cv = stripe[pl.ds(i * L, L)]
            stripe[pl.ds(i * L, L)] = 1.0 / jnp.maximum(cv, 1.0)
            return c
        lax.fori_loop(0, STRIPE // L, nconv, 0)
        pltpu.sync_copy(stripe, csh.at[pl.ds(sid * STRIPE, STRIPE)])

        @pl.when(cid == 0)
        def _():
            pltpu.sync_copy(stripe, norm_out.at[pl.ds(sid * STRIPE, STRIPE)])
        plsc.subcore_barrier()

    wid = cid * NS + sid

    def p3(c, carry):
        base = wid * T3 + c * KC
        pltpu.sync_copy(s_hbm.at[pl.ds(base, KC)], s_b)
        pltpu.sync_copy(r_hbm.at[pl.ds(base, KC)], r_b)
        pltpu.sync_copy(o_hbm.at[pl.ds(base, KC)], o_b)

        def icomp(j, cc):
            blk = j // (B // L)
            off = (j % (B // L)) * L
            sv = s_b[pl.ds(j * L, L)]
            rv = r_b[pl.ds(j * L, L)]
            ov = o_b[pl.ds(j * L, L)]
            a = sv * R + rv
            b = ov * R + rv
            rid2[blk, pl.ds(off, L)] = a
            rid2[NB + blk, pl.ds(off, L)] = b + NREL
            key2[blk, pl.ds(off, L)] = b
            key2[NB + blk, pl.ds(off, L)] = a + NREL
            dst2[blk, pl.ds(off, L)] = ov
            dst2[NB + blk, pl.ds(off, L)] = sv
            return cc
        lax.fori_loop(0, KC // L, icomp, 0)

        ha = []
        for j in range(NB):
            ha.append(pltpu.async_copy(xw_hbm.at[rid2.at[j]], rows.at[j], sem))
            ha.append(pltpu.async_copy(csh.at[key2.at[j]], normv.at[j], sem2))
        hb = []
        for j in range(NB, 2 * NB):
            hb.append(pltpu.async_copy(xw_hbm.at[rid2.at[j]], rows.at[j], sem3))
            hb.append(pltpu.async_copy(csh.at[key2.at[j]], normv.at[j], sem4))

        def scale(g, cc):
            blk = g // (B // L)
            off = (g % (B // L)) * L
            nv = normv[blk, pl.ds(off, L)]
            for i in range(L):
                nrm = nv[i]
                for k in range(H // L):
                    rows[blk, off + i, pl.ds(k * L, L)] = (
                        rows[blk, off + i, pl.ds(k * L, L)] * nrm)
            return cc

        for h in ha:
            h.wait()
        lax.fori_loop(0, KC // L, scale, 0)
        for j in range(NB):
            pltpu.sync_copy(rows.at[j], acc.at[dst2.at[j]], add=True)
        for h in hb:
            h.wait()
        lax.fori_loop(KC // L, (2 * KC) // L, scale, 0)
        for j in range(NB, 2 * NB):
            pltpu.sync_copy(rows.at[j], acc.at[dst2.at[j]], add=True)
        return carry
    lax.fori_loop(0, T3 // KC, p3, 0)
    plsc.subcore_barrier()

    pltpu.sync_copy(acc.at[pl.ds(sid * ROWS_T, ROWS_T), :],
                    acc_out.at[cid, pl.ds(sid * ROWS_T, ROWS_T), :])


def _make_sc_layer(H, compute_counts):
    mesh = plsc.VectorSubcoreMesh(core_axis_name="c", subcore_axis_name="s")
    if compute_counts:
        out_type = [jax.ShapeDtypeStruct((NC, NPAD, H), jnp.float32),
                    jax.ShapeDtypeStruct((CSH,), jnp.float32)]
    else:
        out_type = jax.ShapeDtypeStruct((NC, NPAD, H), jnp.float32)
    scratch = [
        pltpu.VMEM((KC,), jnp.int32),
        pltpu.VMEM((KC,), jnp.int32),
        pltpu.VMEM((KC,), jnp.int32),
        pltpu.VMEM((2 * NB, B), jnp.int32),
        pltpu.VMEM((B,), jnp.float32),
        pltpu.VMEM((2 * NB, B), jnp.int32),
        pltpu.VMEM((2 * NB, B), jnp.int32),
        pltpu.VMEM((2 * NB, B), jnp.int32),
        pltpu.VMEM((2 * NB, B), jnp.float32),
        pltpu.VMEM((2 * NB, B, H), jnp.float32),
        pltpu.VMEM((STRIPE,), jnp.float32),
        pltpu.VMEM_SHARED((CSH,), jnp.float32),
        pltpu.VMEM_SHARED((NPAD, H), jnp.float32),
        pltpu.SemaphoreType.DMA,
        pltpu.SemaphoreType.DMA,
        pltpu.SemaphoreType.DMA,
        pltpu.SemaphoreType.DMA,
    ]
    body = functools.partial(_sc_layer_body, compute_counts, H)
    return pl.kernel(
        body, out_type=out_type, mesh=mesh, scratch_types=scratch,
        compiler_params=pltpu.CompilerParams(use_tc_tiling_on_sc=False))


def _mm_body(x_ref, w_ref, o_ref):
    o_ref[...] = jnp.dot(x_ref[...], w_ref[...], preferred_element_type=jnp.float32)


def _mm(X, Wcat):
    n, f = X.shape
    hh = Wcat.shape[1]
    bn = 1024
    return pl.pallas_call(
        _mm_body,
        grid=(n // bn,),
        in_specs=[pl.BlockSpec((bn, f), lambda i: (i, 0)),
                  pl.BlockSpec((f, hh), lambda i: (0, 0))],
        out_specs=pl.BlockSpec((bn, hh), lambda i: (i, 0)),
        out_shape=jax.ShapeDtypeStruct((n, hh), jnp.float32),
    )(X, Wcat)


def _cmb_mm_body(p_ref, self_ref, b_ref, w_ref, o_ref):
    h = p_ref[0] + p_ref[1] + self_ref[...] + b_ref[...]
    h = jnp.maximum(h, 0.0)
    o_ref[...] = jnp.dot(h, w_ref[...], preferred_element_type=jnp.float32)


def _cmb_mm(parts, selfrows, bvec, Wcat):
    n, f = selfrows.shape
    hh = Wcat.shape[1]
    bn = 1024
    return pl.pallas_call(
        _cmb_mm_body,
        grid=(n // bn,),
        in_specs=[pl.BlockSpec((NC, bn, f), lambda i: (0, i, 0)),
                  pl.BlockSpec((bn, f), lambda i: (i, 0)),
                  pl.BlockSpec((1, f), lambda i: (0, 0)),
                  pl.BlockSpec((f, hh), lambda i: (0, 0))],
        out_specs=pl.BlockSpec((bn, hh), lambda i: (i, 0)),
        out_shape=jax.ShapeDtypeStruct((n, hh), jnp.float32),
    )(parts, selfrows, bvec, Wcat)


def _final_body(p_ref, self_ref, b_ref, o_ref):
    x = p_ref[0] + p_ref[1] + self_ref[...] + b_ref[...]
    m = jnp.max(x, axis=1, keepdims=True)
    e = jnp.exp(x - m)
    ssum = jnp.sum(e, axis=1, keepdims=True)
    o_ref[...] = x - m - jnp.log(ssum)


def _final(parts, selfrows, bvec):
    n, f = selfrows.shape
    bn = 1024
    return pl.pallas_call(
        _final_body,
        grid=(n // bn,),
        in_specs=[pl.BlockSpec((NC, bn, f), lambda i: (0, i, 0)),
                  pl.BlockSpec((bn, f), lambda i: (i, 0)),
                  pl.BlockSpec((1, f), lambda i: (0, 0))],
        out_specs=pl.BlockSpec((bn, f), lambda i: (i, 0)),
        out_shape=jax.ShapeDtypeStruct((n, f), jnp.float32),
    )(parts, selfrows, bvec)


_sc_layer1 = _make_sc_layer(NHID, True)
_sc_layer2 = _make_sc_layer(NCLASS, False)


def kernel(triples, X0, W1, b1, W2, b2):
    s = triples[:, 0]
    r = triples[:, 1]
    o = triples[:, 2]
    npade = EPAD - E
    padn = (N + (jnp.arange(npade, dtype=jnp.int32) % (NPAD - N))).astype(s.dtype)
    sp = jnp.concatenate([s, padn])
    rp = jnp.concatenate([r, jnp.zeros((npade,), s.dtype)])
    op = jnp.concatenate([o, padn])

    X0p = jnp.pad(X0, ((0, NPAD - N), (0, 0)))
    W1cat = W1.transpose(1, 0, 2).reshape(NFEAT, R * NHID)
    W2cat = W2.transpose(1, 0, 2).reshape(NHID, R * NCLASS)

    xw1 = _mm(X0p, W1cat)
    xw1_flat = xw1.reshape(NPAD * R, NHID)
    self1 = xw1.reshape(NPAD, R, NHID)[:, 2 * NREL]

    p1, norm = _sc_layer1(sp, rp, op, xw1_flat)

    hxw2 = _cmb_mm(p1, self1, b1.reshape(1, -1), W2cat)
    xw2_flat = hxw2.reshape(NPAD * R, NCLASS)
    self2 = hxw2.reshape(NPAD, R, NCLASS)[:, 2 * NREL]

    p2 = _sc_layer2(sp, rp, op, xw2_flat, norm)

    out = _final(p2, self2, b2.reshape(1, -1))
    return out[:N]
